# baseline (speedup 1.0000x reference)
import functools

import jax
import jax.numpy as jnp
from jax import lax
from jax.experimental import pallas as pl
from jax.experimental.pallas import tpu as pltpu
from jax.experimental.pallas import tpu_sc as plsc

EMBED_DIM = 64
BATCH = 16384
VOCAB = 14641

_NC, _NS = 2, 16
_NW = _NC * _NS
_NG = EMBED_DIM // 8
_NQ = _NW // _NG
_BPQ = BATCH // _NQ
_CCH = 256
_NCH = _BPQ // _CCH
_L = 16


def _make_gather():
    mesh = plsc.VectorSubcoreMesh(core_axis_name="c", subcore_axis_name="s")

    @functools.partial(
        pl.kernel,
        mesh=mesh,
        out_type=jax.ShapeDtypeStruct((EMBED_DIM, BATCH), jnp.float32),
        scratch_types=[
            pltpu.VMEM((8, VOCAB), jnp.float32),
            pltpu.VMEM((_BPQ,), jnp.int32),
            pltpu.VMEM((2, 8, _CCH), jnp.float32),
            pltpu.SemaphoreType.DMA,
            pltpu.SemaphoreType.DMA,
        ],
        compiler_params=pltpu.CompilerParams(
            use_tc_tiling_on_sc=True,
            needs_layout_passes=False,
            disable_bounds_checks=True,
            disable_semaphore_checks=True,
        ),
    )
    def gather_kernel(tabT_hbm, idx_hbm, outT_hbm, tab_v, idx_v, ob_v,
                      lsem, ssem):
        wid = lax.axis_index("s") * _NC + lax.axis_index("c")
        g = wid % _NG
        q = wid // _NG
        load_tab = pltpu.async_copy(
            tabT_hbm.at[pl.ds(g * 8, 8), :], tab_v, lsem)
        load_idx = pltpu.async_copy(
            idx_hbm.at[pl.ds(q * _BPQ, _BPQ)], idx_v, lsem)
        load_tab.wait()
        load_idx.wait()

        stores = [None, None]
        for ch in range(_NCH):
            buf = ch % 2
            if stores[buf] is not None:
                stores[buf].wait()
            for t in range(_CCH // _L):
                col_idx = idx_v[pl.ds(ch * _CCH + t * _L, _L)]
                for r in range(8):
                    row_idx = jnp.full((_L,), r, dtype=jnp.int32)
                    vals = plsc.load_gather(tab_v, [row_idx, col_idx])
                    ob_v[buf, r, pl.ds(t * _L, _L)] = vals
            stores[buf] = pltpu.async_copy(
                ob_v.at[buf],
                outT_hbm.at[pl.ds(g * 8, 8),
                            pl.ds(q * _BPQ + ch * _CCH, _CCH)],
                ssem,
            )
        for s in stores:
            if s is not None:
                s.wait()

    return gather_kernel


_gather = _make_gather()


def kernel(table, indices):
    return _gather(table.T, indices).T


# speedup vs baseline: 1.2410x; 1.2410x over previous
import functools

import jax
import jax.numpy as jnp
from jax import lax
from jax.experimental import pallas as pl
from jax.experimental.pallas import tpu as pltpu
from jax.experimental.pallas import tpu_sc as plsc

EMBED_DIM = 64
BATCH = 16384
VOCAB = 14641

_NC, _NS = 2, 16
_NW = _NC * _NS
_NG = EMBED_DIM // 8
_NQ = _NW // _NG
_BPQ = BATCH // _NQ
_CCH = 256
_NCH = _BPQ // _CCH
_L = 16


def _make_gather():
    mesh = plsc.VectorSubcoreMesh(core_axis_name="c", subcore_axis_name="s")

    @functools.partial(
        pl.kernel,
        mesh=mesh,
        out_type=jax.ShapeDtypeStruct((EMBED_DIM, BATCH), jnp.float32),
        scratch_types=[
            pltpu.VMEM((8, VOCAB), jnp.float32),
            pltpu.VMEM((_BPQ,), jnp.int32),
            pltpu.VMEM((2, 8, _CCH), jnp.float32),
            pltpu.SemaphoreType.DMA,
            pltpu.SemaphoreType.DMA,
        ],
        compiler_params=pltpu.CompilerParams(
            use_tc_tiling_on_sc=True,
            needs_layout_passes=False,
            disable_bounds_checks=True,
            disable_semaphore_checks=True,
        ),
    )
    def gather_kernel(tabT_hbm, idx_hbm, outT_hbm, tab_v, idx_v, ob_v,
                      lsem, ssem):
        wid = lax.axis_index("s") * _NC + lax.axis_index("c")
        g = wid % _NG
        q = wid // _NG
        load_tab = pltpu.async_copy(
            tabT_hbm.at[pl.ds(g * 8, 8), :], tab_v, lsem)
        load_idx = pltpu.async_copy(
            idx_hbm.at[pl.ds(q * _BPQ, _BPQ)], idx_v, lsem)
        load_tab.wait()
        load_idx.wait()

        row_ids = [jnp.full((_L,), r, dtype=jnp.int32) for r in range(8)]

        def _compute_chunk(ch, buf):
            for t in range(_CCH // _L):
                col_idx = idx_v[pl.ds(ch * _CCH + t * _L, _L)]
                for r in range(8):
                    vals = plsc.load_gather(tab_v, [row_ids[r], col_idx])
                    ob_v[buf, r, pl.ds(t * _L, _L)] = vals

        def _store_desc(ch, buf):
            return pltpu.make_async_copy(
                ob_v.at[buf],
                outT_hbm.at[pl.ds(g * 8, 8),
                            pl.ds(q * _BPQ + ch * _CCH, _CCH)],
                ssem,
            )

        def _pair_body(p, carry):
            for b in range(2):
                ch = p * 2 + b
                pl.when(p > 0)(lambda: _store_desc(ch, b).wait())
                _compute_chunk(ch, b)
                _store_desc(ch, b).start()
            return carry

        lax.fori_loop(0, _NCH // 2, _pair_body, 0)
        for b in range(2):
            _store_desc(_NCH - 2 + b, b).wait()

    return gather_kernel


_gather = _make_gather()


def kernel(table, indices):
    return _gather(table.T, indices).T


# speedup vs baseline: 1.6347x; 1.3172x over previous
import functools

import jax
import jax.numpy as jnp
from jax import lax
from jax.experimental import pallas as pl
from jax.experimental.pallas import tpu as pltpu
from jax.experimental.pallas import tpu_sc as plsc

EMBED_DIM = 64
BATCH = 16384
VOCAB = 14641

_NC, _NS = 2, 16
_NW = _NC * _NS
_NG = EMBED_DIM // 8
_NQ = _NW // _NG
_BPQ = BATCH // _NQ
_CCH = 256
_NCH = _BPQ // _CCH
_L = 16


def _make_gather():
    mesh = plsc.VectorSubcoreMesh(core_axis_name="c", subcore_axis_name="s")

    @functools.partial(
        pl.kernel,
        mesh=mesh,
        out_type=jax.ShapeDtypeStruct((EMBED_DIM, BATCH), jnp.float32),
        scratch_types=[
            pltpu.VMEM((8, VOCAB), jnp.float32),
            pltpu.VMEM((_BPQ,), jnp.int32),
            pltpu.VMEM((2, 8, _CCH), jnp.float32),
            pltpu.SemaphoreType.DMA,
            pltpu.SemaphoreType.DMA,
        ],
        compiler_params=pltpu.CompilerParams(
            use_tc_tiling_on_sc=True,
            needs_layout_passes=False,
            disable_bounds_checks=True,
            disable_semaphore_checks=True,
        ),
    )
    def gather_kernel(tabT_hbm, idx_hbm, outT_hbm, tab_v, idx_v, ob_v,
                      lsem, ssem):
        wid = lax.axis_index("s") * _NC + lax.axis_index("c")
        g = wid % _NG
        q = wid // _NG
        load_tab = pltpu.async_copy(
            tabT_hbm.at[pl.ds(g * 8, 8), :], tab_v, lsem)
        load_idx = pltpu.async_copy(
            idx_hbm.at[pl.ds(q * _BPQ, _BPQ)], idx_v, lsem)
        load_tab.wait()
        load_idx.wait()

        row_ids = [jnp.full((_L,), r, dtype=jnp.int32) for r in range(8)]

        def _compute_chunk(ch, buf):
            @plsc.parallel_loop(0, _CCH // _L, 1, unroll=4)
            def _group(t):
                col_idx = idx_v[pl.ds(ch * _CCH + t * _L, _L)]
                for r in range(8):
                    vals = plsc.load_gather(tab_v, [row_ids[r], col_idx])
                    ob_v[buf, r, pl.ds(t * _L, _L)] = vals

        def _store_desc(ch, buf):
            return pltpu.make_async_copy(
                ob_v.at[buf],
                outT_hbm.at[pl.ds(g * 8, 8),
                            pl.ds(q * _BPQ + ch * _CCH, _CCH)],
                ssem,
            )

        def _pair_body(p, carry):
            for b in range(2):
                ch = p * 2 + b
                pl.when(p > 0)(lambda: _store_desc(ch, b).wait())
                _compute_chunk(ch, b)
                _store_desc(ch, b).start()
            return carry

        lax.fori_loop(0, _NCH // 2, _pair_body, 0)
        for b in range(2):
            _store_desc(_NCH - 2 + b, b).wait()

    return gather_kernel


_gather = _make_gather()


def kernel(table, indices):
    return _gather(table.T, indices).T
